# baseline (speedup 1.0000x reference)
import functools

import jax
import jax.numpy as jnp
from jax import lax
from jax.experimental import pallas as pl
from jax.experimental.pallas import tpu as pltpu
from jax.experimental.pallas import tpu_sc as plsc

NUM_NODES = 1000000
EMBED_DIM = 64
BATCH = 16384

_NC = 2
_NS = 16
_NW = _NC * _NS
_B_PER_W = BATCH // _NW
_CHUNK = 128
_NCHUNK = _B_PER_W // _CHUNK


def _sc_gather(grp_hbm, rem_hbm, pairs_hbm, out_hbm,
               grp_v, rem_v, g_v, out_v, sem0, sem1):
    wid = lax.axis_index("s") * _NC + lax.axis_index("c")
    sems = (sem0, sem1)
    pltpu.sync_copy(grp_hbm.at[wid], grp_v)
    pltpu.sync_copy(rem_hbm.at[wid], rem_v)
    copies = [None, None]
    copies[0] = pltpu.async_copy(pairs_hbm.at[grp_v.at[0]], g_v.at[0], sems[0])
    for j in range(_NCHUNK):
        if j + 1 < _NCHUNK:
            b = (j + 1) % 2
            copies[b] = pltpu.async_copy(
                pairs_hbm.at[grp_v.at[j + 1]], g_v.at[b], sems[b])
        copies[j % 2].wait()
        for g in range(_CHUNK // 16):
            rem16 = rem_v[j, pl.ds(g * 16, 16)]
            for l in range(16):
                i = g * 16 + l
                base = rem16[l] * EMBED_DIM
                for k in range(EMBED_DIM // 16):
                    out_v[j % 2, i, pl.ds(k * 16, 16)] = (
                        g_v[j % 2, i, pl.ds(base + k * 16, 16)])
        pltpu.sync_copy(
            out_v.at[j % 2],
            out_hbm.at[pl.ds(wid * _B_PER_W + j * _CHUNK, _CHUNK)])


@jax.jit
def kernel(batch, emb_weight):
    idx = batch.astype(jnp.int32)
    grp = (idx >> 1).reshape(_NW, _NCHUNK, _CHUNK)
    rem = (idx & 1).reshape(_NW, _NCHUNK, _CHUNK)
    pairs = emb_weight.reshape(NUM_NODES // 2, 2 * EMBED_DIM)
    mesh = plsc.VectorSubcoreMesh(core_axis_name="c", subcore_axis_name="s")
    call = functools.partial(
        pl.kernel,
        mesh=mesh,
        out_type=jax.ShapeDtypeStruct((BATCH, EMBED_DIM), jnp.float32),
        scratch_types=[
            pltpu.VMEM((_NCHUNK, _CHUNK), jnp.int32),
            pltpu.VMEM((_NCHUNK, _CHUNK), jnp.int32),
            pltpu.VMEM((2, _CHUNK, 2 * EMBED_DIM), jnp.float32),
            pltpu.VMEM((2, _CHUNK, EMBED_DIM), jnp.float32),
            pltpu.SemaphoreType.DMA,
            pltpu.SemaphoreType.DMA,
        ],
    )(_sc_gather)
    return call(grp, rem, pairs)
